# baseline (speedup 1.0000x reference)
import jax
import jax.numpy as jnp
from jax import lax
from jax.experimental import pallas as pl
from jax.experimental.pallas import tpu as pltpu
from jax.experimental.pallas import tpu_sc as plsc

N = 50000
E = 800000
G = 256
D = 64
Q = 16
EPS = 1e-5

NP = 50176
ROWS = 6272
EP = ROWS * 128
TROWS = ROWS // 16
GRP = 8
NGRP = TROWS // GRP
DEGSPLIT = 24
R = NP
ZCH = 784
NT = NP // 16
GACC = 1032
PROWS = 4 * NP // 128
NP2 = NP // 2
BLK2 = 1568
GRID = NP2 // BLK2

_mesh = plsc.VectorSubcoreMesh(core_axis_name="c", subcore_axis_name="s")
_sc_params = pltpu.CompilerParams(use_tc_tiling_on_sc=False)


def _make_edge(with_deg, with_pool):
  def body(*refs):
    it = iter(refs)
    src_hbm = next(it)
    dst_hbm = next(it)
    x_hbm = next(it)
    z16 = next(it)
    if with_deg:
      ones16 = next(it)
    if with_pool:
      z8 = next(it)
      gidx_hbm = next(it)
      ones8 = next(it)
    msg_out = next(it)
    if with_deg:
      deg_out = next(it)
    if with_pool:
      hg_out = next(it)
      cnt_out = next(it)
    src_a = next(it)
    dst_a = next(it)
    rows_a = next(it)
    src_b = next(it)
    dst_b = next(it)
    rows_b = next(it)
    acc_sh = next(it)
    sem_ga = next(it)
    sem_gb = next(it)
    if with_deg:
      ones16_v = next(it)
    if with_pool:
      gid_v = next(it)
      chunk_v = next(it)
      ones8_v = next(it)
      gacc_sh = next(it)
      cnt_sh = next(it)

    c = lax.axis_index("c")
    s = lax.axis_index("s")

    def fire(p, g, src_v, dst_v, rows_v, sem):
      r0 = s * TROWS + g * GRP
      pltpu.sync_copy(src_hbm.at[p, c, pl.ds(r0, GRP), :], src_v)
      pltpu.sync_copy(dst_hbm.at[pl.ds(r0, GRP), :], dst_v)
      for j in range(GRP):
        pltpu.async_copy(x_hbm.at[src_v.at[j]], rows_v.at[j], sem)

    def drain_scatter(src_v, dst_v, rows_v, sem):
      for j in range(GRP):
        pltpu.make_async_copy(x_hbm.at[src_v.at[j]], rows_v.at[j],
                              sem).wait()
        pltpu.sync_copy(rows_v.at[j], acc_sh.at[dst_v.at[j]], add=True)

    def zero_acc():
      base = s * NT
      for q in range(4):
        pltpu.sync_copy(z16, acc_sh.at[pl.ds(base + q * ZCH, ZCH), :])

    if with_pool:
      pltpu.sync_copy(z16.at[pl.ds(0, 64), :],
                      gacc_sh.at[pl.ds(s * 64, 64), :])
      pltpu.sync_copy(z8.at[pl.ds(0, 64), :],
                      cnt_sh.at[pl.ds(s * 64, 64), :])
      @pl.when(s == 0)
      def _():
        pltpu.sync_copy(z16.at[pl.ds(64, 8), :],
                        gacc_sh.at[pl.ds(1024, 8), :])
        pltpu.sync_copy(z8.at[pl.ds(64, 8), :],
                        cnt_sh.at[pl.ds(1024, 8), :])
      pltpu.sync_copy(ones8, ones8_v)

    for p in range(2):
      zero_acc()
      plsc.subcore_barrier()

      fire(p, 0, src_a, dst_a, rows_a, sem_ga)

      @pl.loop(0, NGRP // 2)
      def _(k):
        fire(p, 2 * k + 1, src_b, dst_b, rows_b, sem_gb)
        drain_scatter(src_a, dst_a, rows_a, sem_ga)
        fire(p, 2 * k + 2, src_a, dst_a, rows_a, sem_ga)
        drain_scatter(src_b, dst_b, rows_b, sem_gb)

      drain_scatter(src_a, dst_a, rows_a, sem_ga)

      plsc.subcore_barrier()
      pltpu.sync_copy(acc_sh.at[pl.ds(s * NT, NT), :],
                      msg_out.at[pl.ds(s * NT, NT), 2 * p + c, :])

    if with_deg:
      zero_acc()
      pltpu.sync_copy(ones16, ones16_v)
      plsc.subcore_barrier()

      @pl.loop(0, NGRP)
      def _(g):
        @pl.when((g < DEGSPLIT) == (c == 0))
        def _():
          r0 = s * TROWS + g * GRP
          pltpu.sync_copy(dst_hbm.at[pl.ds(r0, GRP), :], dst_a)
          for j in range(GRP):
            pltpu.sync_copy(ones16_v, acc_sh.at[dst_a.at[j]], add=True)

      plsc.subcore_barrier()
      pltpu.sync_copy(acc_sh.at[pl.ds(s * NT, NT), :],
                      deg_out.at[c, pl.ds(s * NT, NT), :])

    if with_pool:
      @pl.loop(0, PROWS // 32)
      def _(k):
        row = c * (PROWS // 2) + s * (PROWS // 32) + k
        pltpu.sync_copy(gidx_hbm.at[row], gid_v)
        pltpu.sync_copy(x_hbm.at[pl.ds(row * 128, 128), :], chunk_v)
        pltpu.sync_copy(chunk_v, gacc_sh.at[gid_v], add=True)
        pltpu.sync_copy(ones8_v, cnt_sh.at[gid_v], add=True)

      plsc.subcore_barrier()
      pltpu.sync_copy(gacc_sh.at[pl.ds(s * 64, 64), :],
                      hg_out.at[c, pl.ds(s * 64, 64), :])
      pltpu.sync_copy(cnt_sh.at[pl.ds(s * 64, 64), :],
                      cnt_out.at[c, pl.ds(s * 64, 64), :])

  out_type = [jax.ShapeDtypeStruct((NP, 4, Q), jnp.float32)]
  if with_deg:
    out_type.append(jax.ShapeDtypeStruct((2, NP, Q), jnp.float32))
  if with_pool:
    out_type.append(jax.ShapeDtypeStruct((2, GACC, Q), jnp.float32))
    out_type.append(jax.ShapeDtypeStruct((2, GACC, 8), jnp.float32))
  scratch = [
      pltpu.VMEM((GRP, 128), jnp.int32),
      pltpu.VMEM((GRP, 128), jnp.int32),
      pltpu.VMEM((GRP, 128, Q), jnp.float32),
      pltpu.VMEM((GRP, 128), jnp.int32),
      pltpu.VMEM((GRP, 128), jnp.int32),
      pltpu.VMEM((GRP, 128, Q), jnp.float32),
      pltpu.VMEM_SHARED((R, Q), jnp.float32),
      pltpu.SemaphoreType.DMA,
      pltpu.SemaphoreType.DMA,
  ]
  if with_deg:
    scratch.append(pltpu.VMEM((128, Q), jnp.float32))
  if with_pool:
    scratch += [
        pltpu.VMEM((128,), jnp.int32),
        pltpu.VMEM((128, Q), jnp.float32),
        pltpu.VMEM((128, 8), jnp.float32),
        pltpu.VMEM_SHARED((GACC, Q), jnp.float32),
        pltpu.VMEM_SHARED((GACC, 8), jnp.float32),
    ]
  return pl.kernel(body, out_type=out_type, mesh=_mesh,
                   scratch_types=scratch, compiler_params=_sc_params)


_edge_deg = _make_edge(True, False)
_edge_pool = _make_edge(False, True)


def _make_pool2():
  def body(e2_hbm, gidx_hbm, z16, hg_out, gid_v, chunk_v, gacc_sh):
    c = lax.axis_index("c")
    s = lax.axis_index("s")
    pltpu.sync_copy(z16.at[pl.ds(0, 64), :], gacc_sh.at[pl.ds(s * 64, 64), :])
    @pl.when(s == 0)
    def _():
      pltpu.sync_copy(z16.at[pl.ds(64, 8), :], gacc_sh.at[pl.ds(1024, 8), :])
    plsc.subcore_barrier()

    @pl.loop(0, PROWS // 32)
    def _(k):
      row = c * (PROWS // 2) + s * (PROWS // 32) + k
      pltpu.sync_copy(gidx_hbm.at[row], gid_v)
      pltpu.sync_copy(e2_hbm.at[pl.ds(row * 128, 128), :], chunk_v)
      pltpu.sync_copy(chunk_v, gacc_sh.at[gid_v], add=True)

    plsc.subcore_barrier()
    pltpu.sync_copy(gacc_sh.at[pl.ds(s * 64, 64), :],
                    hg_out.at[c, pl.ds(s * 64, 64), :])

  return pl.kernel(
      body,
      out_type=jax.ShapeDtypeStruct((2, GACC, Q), jnp.float32),
      mesh=_mesh,
      scratch_types=[
          pltpu.VMEM((128,), jnp.int32),
          pltpu.VMEM((128, Q), jnp.float32),
          pltpu.VMEM_SHARED((GACC, Q), jnp.float32),
      ],
      compiler_params=_sc_params)


_pool2 = _make_pool2()


def _emb_call(x2, w2, b2):
  def body(x_ref, w_ref, b_ref, o_ref):
    o_ref[...] = jnp.dot(x_ref[...], w_ref[...],
                         preferred_element_type=jnp.float32) + b_ref[...]

  return pl.pallas_call(
      body,
      grid=(GRID,),
      in_specs=[pl.BlockSpec((BLK2, 128), lambda i: (i, 0)),
                pl.BlockSpec((128, 128), lambda i: (0, 0)),
                pl.BlockSpec((1, 128), lambda i: (0, 0))],
      out_specs=pl.BlockSpec((BLK2, 128), lambda i: (i, 0)),
      out_shape=jax.ShapeDtypeStruct((NP2, 128), jnp.float32),
  )(x2, w2, b2)


def _layer_a(msg2, degb2, sn2, w2, b2):
  def body(m_ref, da, sn, w_ref, b_ref, o_ref, s_ref, ss_ref):
    inv = 1.0 / jnp.maximum(da[...], 1.0)
    agg = m_ref[...] * inv
    h = jnp.dot(agg, w_ref[...], preferred_element_type=jnp.float32)
    h = jnp.maximum(h + b_ref[...], 0.0) * sn[...]
    o_ref[...] = h

    @pl.when(pl.program_id(0) == 0)
    def _():
      s_ref[...] = jnp.zeros_like(s_ref)
      ss_ref[...] = jnp.zeros_like(ss_ref)

    rowid = (lax.broadcasted_iota(jnp.int32, (BLK2, 1), 0)
             + pl.program_id(0) * BLK2)
    hm = jnp.where(rowid < N // 2, h, 0.0)
    s_ref[...] += jnp.sum(hm, axis=0, keepdims=True)
    ss_ref[...] += jnp.sum(hm * hm, axis=0, keepdims=True)

  return pl.pallas_call(
      body,
      grid=(GRID,),
      in_specs=[pl.BlockSpec((BLK2, 128), lambda i: (i, 0)),
                pl.BlockSpec((BLK2, 128), lambda i: (i, 0)),
                pl.BlockSpec((BLK2, 128), lambda i: (i, 0)),
                pl.BlockSpec((128, 128), lambda i: (0, 0)),
                pl.BlockSpec((1, 128), lambda i: (0, 0))],
      out_specs=[pl.BlockSpec((BLK2, 128), lambda i: (i, 0)),
                 pl.BlockSpec((1, 128), lambda i: (0, 0)),
                 pl.BlockSpec((1, 128), lambda i: (0, 0))],
      out_shape=[jax.ShapeDtypeStruct((NP2, 128), jnp.float32),
                 jax.ShapeDtypeStruct((1, 128), jnp.float32),
                 jax.ShapeDtypeStruct((1, 128), jnp.float32)],
  )(msg2, degb2, sn2, w2, b2)


def _layer_b(h2, x2, ssum, sqsum, gamma, beta):
  def body(h_ref, x_ref, s_ref, ss_ref, g_ref, be_ref, o_ref):
    s = s_ref[...]
    ss = ss_ref[...]
    mu = (s[:, :D] + s[:, D:]) / N
    var = (ss[:, :D] + ss[:, D:]) / N - mu * mu
    scale = g_ref[...] * lax.rsqrt(var + EPS)
    shift = be_ref[...] - mu * scale
    scale2 = jnp.concatenate([scale, scale], axis=1)
    shift2 = jnp.concatenate([shift, shift], axis=1)
    o_ref[...] = x_ref[...] + h_ref[...] * scale2 + shift2

  return pl.pallas_call(
      body,
      grid=(GRID,),
      in_specs=[pl.BlockSpec((BLK2, 128), lambda i: (i, 0)),
                pl.BlockSpec((BLK2, 128), lambda i: (i, 0)),
                pl.BlockSpec((1, 128), lambda i: (0, 0)),
                pl.BlockSpec((1, 128), lambda i: (0, 0)),
                pl.BlockSpec((1, D), lambda i: (0, 0)),
                pl.BlockSpec((1, D), lambda i: (0, 0))],
      out_specs=pl.BlockSpec((BLK2, 128), lambda i: (i, 0)),
      out_shape=jax.ShapeDtypeStruct((NP2, 128), jnp.float32),
  )(h2, x2, ssum, sqsum, gamma, beta)


def _final(hg1p, hg2p, cnt4, w, b):
  def body(*refs):
    parts = refs[:16]
    cnt0, cnt1, w_ref, b_ref, o_ref = refs[16:]
    cnt = cnt0[0, :, 0:1] + cnt1[0, :, 0:1]
    inv = 1.0 / jnp.maximum(cnt, 1.0)
    o = b_ref[...] * jnp.ones((G, 1), jnp.float32)
    for q in range(4):
      sq = (parts[q][0] + parts[4 + q][0] + parts[8 + q][0]
            + parts[12 + q][0]) * inv
      o = o + jnp.dot(sq, w_ref[q * Q:(q + 1) * Q, :],
                      preferred_element_type=jnp.float32)
    o_ref[...] = o

  qspecs = [pl.BlockSpec((1, G, Q), lambda i, c=c, q=q: (c, q, 0))
            for c in range(2) for q in range(4)]
  return pl.pallas_call(
      body,
      grid=(1,),
      in_specs=qspecs + qspecs +
      [pl.BlockSpec((1, G, 8), lambda i: (0, 0, 0)),
       pl.BlockSpec((1, G, 8), lambda i: (1, 0, 0)),
       pl.BlockSpec((D, D), lambda i: (0, 0)),
       pl.BlockSpec((1, D), lambda i: (0, 0))],
      out_specs=pl.BlockSpec((G, D), lambda i: (0, 0)),
      out_shape=jax.ShapeDtypeStruct((G, D), jnp.float32),
  )(*([hg1p] * 8), *([hg2p] * 8), cnt4, cnt4, w, b)


def _pack_w(w, b):
  w2 = jnp.zeros((128, 128), jnp.float32)
  w2 = w2.at[:D, :D].set(w).at[D:, D:].set(w)
  b2 = jnp.concatenate([b, b]).reshape(1, 128)
  return w2, b2


def kernel(nodes_feat, edge_index, edges_feat, nodes_num_norm_sqrt,
           edges_num_norm_sqrt, graph_ids, W_emb, b_emb, W1, b1, gamma1,
           beta1, Wo, bo, gamma_o, beta_o, W_out, b_out):
  src = edge_index[0]
  dst = edge_index[1]
  pad = EP - E
  srcP = 4 * jnp.concatenate([src, jnp.zeros((pad,), jnp.int32)])
  dstP = jnp.concatenate([dst, jnp.full((pad,), N, jnp.int32)])
  src4 = jnp.stack([srcP, srcP + 1, srcP + 2,
                    srcP + 3]).reshape(2, 2, ROWS, 128)
  dst2 = dstP.reshape(ROWS, 128)
  gidQ = 256 * jnp.tile(jnp.arange(4, dtype=jnp.int32), NP)
  gidN = jnp.repeat(
      jnp.concatenate([graph_ids, jnp.full((NP - N,), -1, jnp.int32)]), 4)
  gidx = jnp.where(gidN >= 0, gidQ + gidN,
                   1024 + jnp.tile(jnp.arange(4, dtype=jnp.int32), NP))
  gidx2 = gidx.reshape(PROWS, 128)
  z16 = jnp.zeros((ZCH, Q), jnp.float32)
  z8 = jnp.zeros((ZCH, 8), jnp.float32)
  ones16 = jnp.ones((128, Q), jnp.float32)
  ones8 = jnp.ones((128, 8), jnp.float32)

  nf2 = nodes_feat.reshape(N // 2, 2 * D)
  snP = jnp.concatenate([nodes_num_norm_sqrt,
                         jnp.zeros((NP - N, 1), jnp.float32)])
  sn2 = jnp.broadcast_to(snP, (NP, D)).reshape(NP2, 128)
  w_emb2, b_emb2 = _pack_w(W_emb, b_emb)
  w12, b12 = _pack_w(W1, b1)
  wo2, bo2 = _pack_w(Wo, bo)

  e0 = _emb_call(nf2, w_emb2, b_emb2)
  msg1, deg2 = _edge_deg(src4, dst2, e0.reshape(4 * NP, Q), z16, ones16)
  dv = deg2.reshape(2, NP // 8, 128)
  deg_nodes = (dv[0] + dv[1]).reshape(NP, Q)[:, :1]
  degb2 = jnp.broadcast_to(deg_nodes, (NP, D)).reshape(NP2, 128)
  h1, s1, ss1 = _layer_a(msg1.reshape(NP2, 128), degb2, sn2, w12, b12)
  e1 = _layer_b(h1, e0, s1, ss1, gamma1.reshape(1, D), beta1.reshape(1, D))
  e1v = e1.reshape(4 * NP, Q)
  msg2, hg1p, cnt4 = _edge_pool(src4, dst2, e1v, z16, z8, gidx2, ones8)
  h2, s2, ss2 = _layer_a(msg2.reshape(NP2, 128), degb2, sn2, wo2, bo2)
  e2 = _layer_b(h2, e1, s2, ss2, gamma_o.reshape(1, D),
                beta_o.reshape(1, D))
  hg2p = _pool2(e2.reshape(4 * NP, Q), gidx2, z16)
  return _final(hg1p, hg2p, cnt4, W_out, b_out.reshape(1, D))


# speedup vs baseline: 1.0360x; 1.0360x over previous
import jax
import jax.numpy as jnp
from jax import lax
from jax.experimental import pallas as pl
from jax.experimental.pallas import tpu as pltpu
from jax.experimental.pallas import tpu_sc as plsc

N = 50000
E = 800000
G = 256
D = 64
Q = 16
EPS = 1e-5

NP = 50176
ROWS = 6272
EP = ROWS * 128
TROWS = ROWS // 16
GRP = 8
NGRP = TROWS // GRP
DEGSPLIT = 24
R = NP
ZCH = 784
NT = NP // 16
GACC = 1032
PROWS = 4 * NP // 128
NP2 = NP // 2
BLK2 = 1568
GRID = NP2 // BLK2

_mesh = plsc.VectorSubcoreMesh(core_axis_name="c", subcore_axis_name="s")
_sc_params = pltpu.CompilerParams(use_tc_tiling_on_sc=False)


def _make_edge(with_deg, with_pool):
  def body(*refs):
    it = iter(refs)
    src_hbm = next(it)
    dst_hbm = next(it)
    x_hbm = next(it)
    z16 = next(it)
    if with_deg:
      ones16 = next(it)
    if with_pool:
      z8 = next(it)
      gidx_hbm = next(it)
      ones8 = next(it)
    msg_out = next(it)
    if with_deg:
      deg_out = next(it)
    if with_pool:
      hg_out = next(it)
      cnt_out = next(it)
    src_a = next(it)
    dst_a = next(it)
    rows_a = next(it)
    src_b = next(it)
    dst_b = next(it)
    rows_b = next(it)
    acc_sh = next(it)
    sem_ga = next(it)
    sem_gb = next(it)
    if with_deg:
      ones16_v = next(it)
    if with_pool:
      gid_v = next(it)
      chunk_v = next(it)
      ones8_v = next(it)
      gacc_sh = next(it)
      cnt_sh = next(it)

    c = lax.axis_index("c")
    s = lax.axis_index("s")

    def fire(p, g, src_v, dst_v, rows_v, sem):
      r0 = s * TROWS + g * GRP
      pltpu.sync_copy(src_hbm.at[p, c, pl.ds(r0, GRP), :], src_v)
      pltpu.sync_copy(dst_hbm.at[pl.ds(r0, GRP), :], dst_v)
      for j in range(GRP):
        pltpu.async_copy(x_hbm.at[src_v.at[j]], rows_v.at[j], sem)

    def drain_scatter(src_v, dst_v, rows_v, sem):
      for j in range(GRP):
        pltpu.make_async_copy(x_hbm.at[src_v.at[j]], rows_v.at[j],
                              sem).wait()
        pltpu.sync_copy(rows_v.at[j], acc_sh.at[dst_v.at[j]], add=True)

    def zero_acc():
      base = s * NT
      for q in range(4):
        pltpu.sync_copy(z16, acc_sh.at[pl.ds(base + q * ZCH, ZCH), :])

    if with_pool:
      pltpu.sync_copy(z16.at[pl.ds(0, 64), :],
                      gacc_sh.at[pl.ds(s * 64, 64), :])
      pltpu.sync_copy(z8.at[pl.ds(0, 64), :],
                      cnt_sh.at[pl.ds(s * 64, 64), :])
      @pl.when(s == 0)
      def _():
        pltpu.sync_copy(z16.at[pl.ds(64, 8), :],
                        gacc_sh.at[pl.ds(1024, 8), :])
        pltpu.sync_copy(z8.at[pl.ds(64, 8), :],
                        cnt_sh.at[pl.ds(1024, 8), :])
      pltpu.sync_copy(ones8, ones8_v)

    for p in range(2):
      zero_acc()
      plsc.subcore_barrier()

      fire(p, 0, src_a, dst_a, rows_a, sem_ga)

      @pl.loop(0, NGRP // 2)
      def _(k):
        fire(p, 2 * k + 1, src_b, dst_b, rows_b, sem_gb)
        drain_scatter(src_a, dst_a, rows_a, sem_ga)
        fire(p, 2 * k + 2, src_a, dst_a, rows_a, sem_ga)
        drain_scatter(src_b, dst_b, rows_b, sem_gb)

      drain_scatter(src_a, dst_a, rows_a, sem_ga)

      plsc.subcore_barrier()
      pltpu.sync_copy(acc_sh.at[pl.ds(s * NT, NT), :],
                      msg_out.at[pl.ds(s * NT, NT), 2 * p + c, :])

    if with_deg:
      zero_acc()
      pltpu.sync_copy(ones16, ones16_v)
      plsc.subcore_barrier()

      @pl.loop(0, NGRP)
      def _(g):
        @pl.when((g < DEGSPLIT) == (c == 0))
        def _():
          r0 = s * TROWS + g * GRP
          pltpu.sync_copy(dst_hbm.at[pl.ds(r0, GRP), :], dst_a)
          for j in range(GRP):
            pltpu.sync_copy(ones16_v, acc_sh.at[dst_a.at[j]], add=True)

      plsc.subcore_barrier()
      for rep in range(4):
        pltpu.sync_copy(acc_sh.at[pl.ds(s * NT, NT), :],
                        deg_out.at[c, pl.ds(s * NT, NT), rep, :])

    if with_pool:
      @pl.loop(0, PROWS // 32)
      def _(k):
        row = c * (PROWS // 2) + s * (PROWS // 32) + k
        pltpu.sync_copy(gidx_hbm.at[row], gid_v)
        pltpu.sync_copy(x_hbm.at[pl.ds(row * 128, 128), :], chunk_v)
        pltpu.sync_copy(chunk_v, gacc_sh.at[gid_v], add=True)
        pltpu.sync_copy(ones8_v, cnt_sh.at[gid_v], add=True)

      plsc.subcore_barrier()
      pltpu.sync_copy(gacc_sh.at[pl.ds(s * 64, 64), :],
                      hg_out.at[c, pl.ds(s * 64, 64), :])
      pltpu.sync_copy(cnt_sh.at[pl.ds(s * 64, 64), :],
                      cnt_out.at[c, pl.ds(s * 64, 64), :])

  out_type = [jax.ShapeDtypeStruct((NP, 4, Q), jnp.float32)]
  if with_deg:
    out_type.append(jax.ShapeDtypeStruct((2, NP, 4, Q), jnp.float32))
  if with_pool:
    out_type.append(jax.ShapeDtypeStruct((2, GACC, Q), jnp.float32))
    out_type.append(jax.ShapeDtypeStruct((2, GACC, 8), jnp.float32))
  scratch = [
      pltpu.VMEM((GRP, 128), jnp.int32),
      pltpu.VMEM((GRP, 128), jnp.int32),
      pltpu.VMEM((GRP, 128, Q), jnp.float32),
      pltpu.VMEM((GRP, 128), jnp.int32),
      pltpu.VMEM((GRP, 128), jnp.int32),
      pltpu.VMEM((GRP, 128, Q), jnp.float32),
      pltpu.VMEM_SHARED((R, Q), jnp.float32),
      pltpu.SemaphoreType.DMA,
      pltpu.SemaphoreType.DMA,
  ]
  if with_deg:
    scratch.append(pltpu.VMEM((128, Q), jnp.float32))
  if with_pool:
    scratch += [
        pltpu.VMEM((128,), jnp.int32),
        pltpu.VMEM((128, Q), jnp.float32),
        pltpu.VMEM((128, 8), jnp.float32),
        pltpu.VMEM_SHARED((GACC, Q), jnp.float32),
        pltpu.VMEM_SHARED((GACC, 8), jnp.float32),
    ]
  return pl.kernel(body, out_type=out_type, mesh=_mesh,
                   scratch_types=scratch, compiler_params=_sc_params)


_edge_deg = _make_edge(True, False)
_edge_pool = _make_edge(False, True)


def _make_pool2():
  def body(e2_hbm, gidx_hbm, z16, hg_out, gid_v, chunk_v, gacc_sh):
    c = lax.axis_index("c")
    s = lax.axis_index("s")
    pltpu.sync_copy(z16.at[pl.ds(0, 64), :], gacc_sh.at[pl.ds(s * 64, 64), :])
    @pl.when(s == 0)
    def _():
      pltpu.sync_copy(z16.at[pl.ds(64, 8), :], gacc_sh.at[pl.ds(1024, 8), :])
    plsc.subcore_barrier()

    @pl.loop(0, PROWS // 32)
    def _(k):
      row = c * (PROWS // 2) + s * (PROWS // 32) + k
      pltpu.sync_copy(gidx_hbm.at[row], gid_v)
      pltpu.sync_copy(e2_hbm.at[pl.ds(row * 128, 128), :], chunk_v)
      pltpu.sync_copy(chunk_v, gacc_sh.at[gid_v], add=True)

    plsc.subcore_barrier()
    pltpu.sync_copy(gacc_sh.at[pl.ds(s * 64, 64), :],
                    hg_out.at[c, pl.ds(s * 64, 64), :])

  return pl.kernel(
      body,
      out_type=jax.ShapeDtypeStruct((2, GACC, Q), jnp.float32),
      mesh=_mesh,
      scratch_types=[
          pltpu.VMEM((128,), jnp.int32),
          pltpu.VMEM((128, Q), jnp.float32),
          pltpu.VMEM_SHARED((GACC, Q), jnp.float32),
      ],
      compiler_params=_sc_params)


_pool2 = _make_pool2()


def _emb_call(x2, w2, b2):
  def body(x_ref, w_ref, b_ref, o_ref):
    o_ref[...] = jnp.dot(x_ref[...], w_ref[...],
                         preferred_element_type=jnp.float32) + b_ref[...]

  return pl.pallas_call(
      body,
      grid=(GRID,),
      in_specs=[pl.BlockSpec((BLK2, 128), lambda i: (i, 0)),
                pl.BlockSpec((128, 128), lambda i: (0, 0)),
                pl.BlockSpec((1, 128), lambda i: (0, 0))],
      out_specs=pl.BlockSpec((BLK2, 128), lambda i: (i, 0)),
      out_shape=jax.ShapeDtypeStruct((NP2, 128), jnp.float32),
  )(x2, w2, b2)


def _layer_a(msg2, dega, degb, sn2, w2, b2):
  def body(m_ref, da, db, sn, w_ref, b_ref, o_ref, s_ref, ss_ref):
    inv = 1.0 / jnp.maximum(da[0] + db[0], 1.0)
    agg = m_ref[...] * inv
    h = jnp.dot(agg, w_ref[...], preferred_element_type=jnp.float32)
    h = jnp.maximum(h + b_ref[...], 0.0) * sn[...]
    o_ref[...] = h

    @pl.when(pl.program_id(0) == 0)
    def _():
      s_ref[...] = jnp.zeros_like(s_ref)
      ss_ref[...] = jnp.zeros_like(ss_ref)

    rowid = (lax.broadcasted_iota(jnp.int32, (BLK2, 1), 0)
             + pl.program_id(0) * BLK2)
    hm = jnp.where(rowid < N // 2, h, 0.0)
    s_ref[...] += jnp.sum(hm, axis=0, keepdims=True)
    ss_ref[...] += jnp.sum(hm * hm, axis=0, keepdims=True)

  return pl.pallas_call(
      body,
      grid=(GRID,),
      in_specs=[pl.BlockSpec((BLK2, 128), lambda i: (i, 0)),
                pl.BlockSpec((1, BLK2, 128), lambda i: (0, i, 0)),
                pl.BlockSpec((1, BLK2, 128), lambda i: (1, i, 0)),
                pl.BlockSpec((BLK2, 128), lambda i: (i, 0)),
                pl.BlockSpec((128, 128), lambda i: (0, 0)),
                pl.BlockSpec((1, 128), lambda i: (0, 0))],
      out_specs=[pl.BlockSpec((BLK2, 128), lambda i: (i, 0)),
                 pl.BlockSpec((1, 128), lambda i: (0, 0)),
                 pl.BlockSpec((1, 128), lambda i: (0, 0))],
      out_shape=[jax.ShapeDtypeStruct((NP2, 128), jnp.float32),
                 jax.ShapeDtypeStruct((1, 128), jnp.float32),
                 jax.ShapeDtypeStruct((1, 128), jnp.float32)],
  )(msg2, dega, degb, sn2, w2, b2)


def _layer_b(h2, x2, ssum, sqsum, gamma, beta):
  def body(h_ref, x_ref, s_ref, ss_ref, g_ref, be_ref, o_ref):
    s = s_ref[...]
    ss = ss_ref[...]
    mu = (s[:, :D] + s[:, D:]) / N
    var = (ss[:, :D] + ss[:, D:]) / N - mu * mu
    scale = g_ref[...] * lax.rsqrt(var + EPS)
    shift = be_ref[...] - mu * scale
    scale2 = jnp.concatenate([scale, scale], axis=1)
    shift2 = jnp.concatenate([shift, shift], axis=1)
    o_ref[...] = x_ref[...] + h_ref[...] * scale2 + shift2

  return pl.pallas_call(
      body,
      grid=(GRID,),
      in_specs=[pl.BlockSpec((BLK2, 128), lambda i: (i, 0)),
                pl.BlockSpec((BLK2, 128), lambda i: (i, 0)),
                pl.BlockSpec((1, 128), lambda i: (0, 0)),
                pl.BlockSpec((1, 128), lambda i: (0, 0)),
                pl.BlockSpec((1, D), lambda i: (0, 0)),
                pl.BlockSpec((1, D), lambda i: (0, 0))],
      out_specs=pl.BlockSpec((BLK2, 128), lambda i: (i, 0)),
      out_shape=jax.ShapeDtypeStruct((NP2, 128), jnp.float32),
  )(h2, x2, ssum, sqsum, gamma, beta)


def _final(hg1p, hg2p, cnt4, w, b):
  def body(*refs):
    parts = refs[:16]
    cnt0, cnt1, w_ref, b_ref, o_ref = refs[16:]
    cnt = cnt0[0, :, 0:1] + cnt1[0, :, 0:1]
    inv = 1.0 / jnp.maximum(cnt, 1.0)
    o = b_ref[...] * jnp.ones((G, 1), jnp.float32)
    for q in range(4):
      sq = (parts[q][0] + parts[4 + q][0] + parts[8 + q][0]
            + parts[12 + q][0]) * inv
      o = o + jnp.dot(sq, w_ref[q * Q:(q + 1) * Q, :],
                      preferred_element_type=jnp.float32)
    o_ref[...] = o

  qspecs = [pl.BlockSpec((1, G, Q), lambda i, c=c, q=q: (c, q, 0))
            for c in range(2) for q in range(4)]
  return pl.pallas_call(
      body,
      grid=(1,),
      in_specs=qspecs + qspecs +
      [pl.BlockSpec((1, G, 8), lambda i: (0, 0, 0)),
       pl.BlockSpec((1, G, 8), lambda i: (1, 0, 0)),
       pl.BlockSpec((D, D), lambda i: (0, 0)),
       pl.BlockSpec((1, D), lambda i: (0, 0))],
      out_specs=pl.BlockSpec((G, D), lambda i: (0, 0)),
      out_shape=jax.ShapeDtypeStruct((G, D), jnp.float32),
  )(*([hg1p] * 8), *([hg2p] * 8), cnt4, cnt4, w, b)


def _pack_w(w, b):
  w2 = jnp.zeros((128, 128), jnp.float32)
  w2 = w2.at[:D, :D].set(w).at[D:, D:].set(w)
  b2 = jnp.concatenate([b, b]).reshape(1, 128)
  return w2, b2


def kernel(nodes_feat, edge_index, edges_feat, nodes_num_norm_sqrt,
           edges_num_norm_sqrt, graph_ids, W_emb, b_emb, W1, b1, gamma1,
           beta1, Wo, bo, gamma_o, beta_o, W_out, b_out):
  src = edge_index[0]
  dst = edge_index[1]
  pad = EP - E
  srcP = 4 * jnp.concatenate([src, jnp.zeros((pad,), jnp.int32)])
  dstP = jnp.concatenate([dst, jnp.full((pad,), N, jnp.int32)])
  src4 = jnp.stack([srcP, srcP + 1, srcP + 2,
                    srcP + 3]).reshape(2, 2, ROWS, 128)
  dst2 = dstP.reshape(ROWS, 128)
  gidQ = 256 * jnp.tile(jnp.arange(4, dtype=jnp.int32), NP)
  gidN = jnp.repeat(
      jnp.concatenate([graph_ids, jnp.full((NP - N,), -1, jnp.int32)]), 4)
  gidx = jnp.where(gidN >= 0, gidQ + gidN,
                   1024 + jnp.tile(jnp.arange(4, dtype=jnp.int32), NP))
  gidx2 = gidx.reshape(PROWS, 128)
  z16 = jnp.zeros((ZCH, Q), jnp.float32)
  z8 = jnp.zeros((ZCH, 8), jnp.float32)
  ones16 = jnp.ones((128, Q), jnp.float32)
  ones8 = jnp.ones((128, 8), jnp.float32)

  nf2 = nodes_feat.reshape(N // 2, 2 * D)
  snP = jnp.concatenate([nodes_num_norm_sqrt,
                         jnp.zeros((NP - N, 1), jnp.float32)])
  sn2 = jnp.broadcast_to(snP, (NP, D)).reshape(NP2, 128)
  w_emb2, b_emb2 = _pack_w(W_emb, b_emb)
  w12, b12 = _pack_w(W1, b1)
  wo2, bo2 = _pack_w(Wo, bo)

  e0 = _emb_call(nf2, w_emb2, b_emb2)
  msg1, deg2 = _edge_deg(src4, dst2, e0.reshape(4 * NP, Q), z16, ones16)
  degp = deg2.reshape(2, NP2, 128)
  h1, s1, ss1 = _layer_a(msg1.reshape(NP2, 128), degp, degp, sn2, w12, b12)
  e1 = _layer_b(h1, e0, s1, ss1, gamma1.reshape(1, D), beta1.reshape(1, D))
  e1v = e1.reshape(4 * NP, Q)
  msg2, hg1p, cnt4 = _edge_pool(src4, dst2, e1v, z16, z8, gidx2, ones8)
  h2, s2, ss2 = _layer_a(msg2.reshape(NP2, 128), degp, degp, sn2, wo2, bo2)
  e2 = _layer_b(h2, e1, s2, ss2, gamma_o.reshape(1, D),
                beta_o.reshape(1, D))
  hg2p = _pool2(e2.reshape(4 * NP, Q), gidx2, z16)
  return _final(hg1p, hg2p, cnt4, W_out, b_out.reshape(1, D))


# speedup vs baseline: 1.0913x; 1.0534x over previous
import jax
import jax.numpy as jnp
from jax import lax
from jax.experimental import pallas as pl
from jax.experimental.pallas import tpu as pltpu
from jax.experimental.pallas import tpu_sc as plsc

N = 50000
E = 800000
G = 256
D = 64
Q = 16
EPS = 1e-5

NP = 50176
ROWS = 6272
EP = ROWS * 128
TROWS = ROWS // 16
GRP = 8
NGRP = TROWS // GRP
DEGSPLIT = 24
R = NP
ZCH = 784
NT = NP // 16
GACC = 1032
PROWS = 4 * NP // 128
NP2 = NP // 2
BLK2 = 1568
GRID = NP2 // BLK2

_mesh = plsc.VectorSubcoreMesh(core_axis_name="c", subcore_axis_name="s")
_sc_params = pltpu.CompilerParams(use_tc_tiling_on_sc=False)


def _make_edge(with_deg, with_pool):
  def body(*refs):
    it = iter(refs)
    src_hbm = next(it)
    dst_hbm = next(it)
    x_hbm = next(it)
    z16 = next(it)
    if with_deg:
      ones16 = next(it)
    if with_pool:
      z8 = next(it)
      gidx_hbm = next(it)
      ones8 = next(it)
    msg_out = next(it)
    if with_deg:
      deg_out = next(it)
    if with_pool:
      hg_out = next(it)
      cnt_out = next(it)
    src_a = next(it)
    dst_a = next(it)
    rows_a = next(it)
    src_b = next(it)
    dst_b = next(it)
    rows_b = next(it)
    acc_sh = next(it)
    sem_ga = next(it)
    sem_gb = next(it)
    if with_deg:
      ones16_v = next(it)
    if with_pool:
      gid_v = next(it)
      chunk_v = next(it)
      gid_b = next(it)
      chunk_b = next(it)
      ones8_v = next(it)
      gacc_sh = next(it)
      cnt_sh = next(it)

    c = lax.axis_index("c")
    s = lax.axis_index("s")

    def fire(p, g, src_v, dst_v, rows_v, sem):
      r0 = s * TROWS + g * GRP
      pltpu.sync_copy(src_hbm.at[p, c, pl.ds(r0, GRP), :], src_v)
      pltpu.sync_copy(dst_hbm.at[pl.ds(r0, GRP), :], dst_v)
      for j in range(GRP):
        pltpu.async_copy(x_hbm.at[src_v.at[j]], rows_v.at[j], sem)

    def drain_scatter(src_v, dst_v, rows_v, sem):
      for j in range(GRP):
        pltpu.make_async_copy(x_hbm.at[src_v.at[j]], rows_v.at[j],
                              sem).wait()
        pltpu.sync_copy(rows_v.at[j], acc_sh.at[dst_v.at[j]], add=True)

    def zero_acc():
      base = s * NT
      for q in range(4):
        pltpu.sync_copy(z16, acc_sh.at[pl.ds(base + q * ZCH, ZCH), :])

    if with_pool:
      pltpu.sync_copy(z16.at[pl.ds(0, 64), :],
                      gacc_sh.at[pl.ds(s * 64, 64), :])
      pltpu.sync_copy(z8.at[pl.ds(0, 64), :],
                      cnt_sh.at[pl.ds(s * 64, 64), :])
      @pl.when(s == 0)
      def _():
        pltpu.sync_copy(z16.at[pl.ds(64, 8), :],
                        gacc_sh.at[pl.ds(1024, 8), :])
        pltpu.sync_copy(z8.at[pl.ds(64, 8), :],
                        cnt_sh.at[pl.ds(1024, 8), :])
      pltpu.sync_copy(ones8, ones8_v)

    for p in range(2):
      zero_acc()
      plsc.subcore_barrier()

      fire(p, 0, src_a, dst_a, rows_a, sem_ga)

      @pl.loop(0, NGRP // 2)
      def _(k):
        fire(p, 2 * k + 1, src_b, dst_b, rows_b, sem_gb)
        drain_scatter(src_a, dst_a, rows_a, sem_ga)
        fire(p, 2 * k + 2, src_a, dst_a, rows_a, sem_ga)
        drain_scatter(src_b, dst_b, rows_b, sem_gb)

      drain_scatter(src_a, dst_a, rows_a, sem_ga)

      plsc.subcore_barrier()
      pltpu.sync_copy(acc_sh.at[pl.ds(s * NT, NT), :],
                      msg_out.at[pl.ds(s * NT, NT), 2 * p + c, :])

    if with_deg:
      zero_acc()
      pltpu.sync_copy(ones16, ones16_v)
      plsc.subcore_barrier()

      @pl.loop(0, NGRP)
      def _(g):
        @pl.when((g < DEGSPLIT) == (c == 0))
        def _():
          r0 = s * TROWS + g * GRP
          pltpu.sync_copy(dst_hbm.at[pl.ds(r0, GRP), :], dst_a)
          for j in range(GRP):
            pltpu.sync_copy(ones16_v, acc_sh.at[dst_a.at[j]], add=True)

      plsc.subcore_barrier()
      for rep in range(4):
        pltpu.sync_copy(acc_sh.at[pl.ds(s * NT, NT), :],
                        deg_out.at[c, pl.ds(s * NT, NT), rep, :])

    if with_pool:
      def pfire(k, gv, chv, sem):
        row = c * (PROWS // 2) + s * (PROWS // 32) + k
        pltpu.sync_copy(gidx_hbm.at[row], gv)
        pltpu.async_copy(x_hbm.at[pl.ds(row * 128, 128), :], chv, sem)

      def pdrain(gv, chv, sem):
        pltpu.make_async_copy(x_hbm.at[pl.ds(0, 128), :], chv, sem).wait()
        pltpu.sync_copy(chv, gacc_sh.at[gv], add=True)
        pltpu.sync_copy(ones8_v, cnt_sh.at[gv], add=True)

      pfire(0, gid_v, chunk_v, sem_ga)

      @pl.loop(0, PROWS // 64)
      def _(k):
        pfire(2 * k + 1, gid_b, chunk_b, sem_gb)
        pdrain(gid_v, chunk_v, sem_ga)
        pfire(2 * k + 2, gid_v, chunk_v, sem_ga)
        pdrain(gid_b, chunk_b, sem_gb)

      pdrain(gid_v, chunk_v, sem_ga)

      plsc.subcore_barrier()
      pltpu.sync_copy(gacc_sh.at[pl.ds(s * 64, 64), :],
                      hg_out.at[c, pl.ds(s * 64, 64), :])
      pltpu.sync_copy(cnt_sh.at[pl.ds(s * 64, 64), :],
                      cnt_out.at[c, pl.ds(s * 64, 64), :])

  out_type = [jax.ShapeDtypeStruct((NP, 4, Q), jnp.float32)]
  if with_deg:
    out_type.append(jax.ShapeDtypeStruct((2, NP, 4, Q), jnp.float32))
  if with_pool:
    out_type.append(jax.ShapeDtypeStruct((2, GACC, Q), jnp.float32))
    out_type.append(jax.ShapeDtypeStruct((2, GACC, 8), jnp.float32))
  scratch = [
      pltpu.VMEM((GRP, 128), jnp.int32),
      pltpu.VMEM((GRP, 128), jnp.int32),
      pltpu.VMEM((GRP, 128, Q), jnp.float32),
      pltpu.VMEM((GRP, 128), jnp.int32),
      pltpu.VMEM((GRP, 128), jnp.int32),
      pltpu.VMEM((GRP, 128, Q), jnp.float32),
      pltpu.VMEM_SHARED((R, Q), jnp.float32),
      pltpu.SemaphoreType.DMA,
      pltpu.SemaphoreType.DMA,
  ]
  if with_deg:
    scratch.append(pltpu.VMEM((128, Q), jnp.float32))
  if with_pool:
    scratch += [
        pltpu.VMEM((128,), jnp.int32),
        pltpu.VMEM((128, Q), jnp.float32),
        pltpu.VMEM((128,), jnp.int32),
        pltpu.VMEM((128, Q), jnp.float32),
        pltpu.VMEM((128, 8), jnp.float32),
        pltpu.VMEM_SHARED((GACC, Q), jnp.float32),
        pltpu.VMEM_SHARED((GACC, 8), jnp.float32),
    ]
  return pl.kernel(body, out_type=out_type, mesh=_mesh,
                   scratch_types=scratch, compiler_params=_sc_params)


_edge_deg = _make_edge(True, False)
_edge_pool = _make_edge(False, True)


def _make_pool2():
  def body(e2_hbm, gidx_hbm, z16, hg_out, gid_v, chunk_v, gid_b, chunk_b,
           gacc_sh, sem_a, sem_b):
    c = lax.axis_index("c")
    s = lax.axis_index("s")
    pltpu.sync_copy(z16.at[pl.ds(0, 64), :], gacc_sh.at[pl.ds(s * 64, 64), :])
    @pl.when(s == 0)
    def _():
      pltpu.sync_copy(z16.at[pl.ds(64, 8), :], gacc_sh.at[pl.ds(1024, 8), :])
    plsc.subcore_barrier()

    def pfire(k, gv, chv, sem):
      row = c * (PROWS // 2) + s * (PROWS // 32) + k
      pltpu.sync_copy(gidx_hbm.at[row], gv)
      pltpu.async_copy(e2_hbm.at[pl.ds(row * 128, 128), :], chv, sem)

    def pdrain(gv, chv, sem):
      pltpu.make_async_copy(e2_hbm.at[pl.ds(0, 128), :], chv, sem).wait()
      pltpu.sync_copy(chv, gacc_sh.at[gv], add=True)

    pfire(0, gid_v, chunk_v, sem_a)

    @pl.loop(0, PROWS // 64)
    def _(k):
      pfire(2 * k + 1, gid_b, chunk_b, sem_b)
      pdrain(gid_v, chunk_v, sem_a)
      pfire(2 * k + 2, gid_v, chunk_v, sem_a)
      pdrain(gid_b, chunk_b, sem_b)

    pdrain(gid_v, chunk_v, sem_a)

    plsc.subcore_barrier()
    pltpu.sync_copy(gacc_sh.at[pl.ds(s * 64, 64), :],
                    hg_out.at[c, pl.ds(s * 64, 64), :])

  return pl.kernel(
      body,
      out_type=jax.ShapeDtypeStruct((2, GACC, Q), jnp.float32),
      mesh=_mesh,
      scratch_types=[
          pltpu.VMEM((128,), jnp.int32),
          pltpu.VMEM((128, Q), jnp.float32),
          pltpu.VMEM((128,), jnp.int32),
          pltpu.VMEM((128, Q), jnp.float32),
          pltpu.VMEM_SHARED((GACC, Q), jnp.float32),
          pltpu.SemaphoreType.DMA,
          pltpu.SemaphoreType.DMA,
      ],
      compiler_params=_sc_params)


_pool2 = _make_pool2()


def _emb_call(x2, w2, b2):
  def body(x_ref, w_ref, b_ref, o_ref):
    o_ref[...] = jnp.dot(x_ref[...], w_ref[...],
                         preferred_element_type=jnp.float32) + b_ref[...]

  return pl.pallas_call(
      body,
      grid=(GRID,),
      in_specs=[pl.BlockSpec((BLK2, 128), lambda i: (i, 0)),
                pl.BlockSpec((128, 128), lambda i: (0, 0)),
                pl.BlockSpec((1, 128), lambda i: (0, 0))],
      out_specs=pl.BlockSpec((BLK2, 128), lambda i: (i, 0)),
      out_shape=jax.ShapeDtypeStruct((NP2, 128), jnp.float32),
  )(x2, w2, b2)


def _layer_a(msg2, dega, degb, sn2, w2, b2):
  def body(m_ref, da, db, sn, w_ref, b_ref, o_ref, s_ref, ss_ref):
    inv = 1.0 / jnp.maximum(da[0] + db[0], 1.0)
    agg = m_ref[...] * inv
    h = jnp.dot(agg, w_ref[...], preferred_element_type=jnp.float32)
    h = jnp.maximum(h + b_ref[...], 0.0) * sn[...]
    o_ref[...] = h

    @pl.when(pl.program_id(0) == 0)
    def _():
      s_ref[...] = jnp.zeros_like(s_ref)
      ss_ref[...] = jnp.zeros_like(ss_ref)

    rowid = (lax.broadcasted_iota(jnp.int32, (BLK2, 1), 0)
             + pl.program_id(0) * BLK2)
    hm = jnp.where(rowid < N // 2, h, 0.0)
    s_ref[...] += jnp.sum(hm, axis=0, keepdims=True)
    ss_ref[...] += jnp.sum(hm * hm, axis=0, keepdims=True)

  return pl.pallas_call(
      body,
      grid=(GRID,),
      in_specs=[pl.BlockSpec((BLK2, 128), lambda i: (i, 0)),
                pl.BlockSpec((1, BLK2, 128), lambda i: (0, i, 0)),
                pl.BlockSpec((1, BLK2, 128), lambda i: (1, i, 0)),
                pl.BlockSpec((BLK2, 128), lambda i: (i, 0)),
                pl.BlockSpec((128, 128), lambda i: (0, 0)),
                pl.BlockSpec((1, 128), lambda i: (0, 0))],
      out_specs=[pl.BlockSpec((BLK2, 128), lambda i: (i, 0)),
                 pl.BlockSpec((1, 128), lambda i: (0, 0)),
                 pl.BlockSpec((1, 128), lambda i: (0, 0))],
      out_shape=[jax.ShapeDtypeStruct((NP2, 128), jnp.float32),
                 jax.ShapeDtypeStruct((1, 128), jnp.float32),
                 jax.ShapeDtypeStruct((1, 128), jnp.float32)],
  )(msg2, dega, degb, sn2, w2, b2)


def _layer_b(h2, x2, ssum, sqsum, gamma, beta):
  def body(h_ref, x_ref, s_ref, ss_ref, g_ref, be_ref, o_ref):
    s = s_ref[...]
    ss = ss_ref[...]
    mu = (s[:, :D] + s[:, D:]) / N
    var = (ss[:, :D] + ss[:, D:]) / N - mu * mu
    scale = g_ref[...] * lax.rsqrt(var + EPS)
    shift = be_ref[...] - mu * scale
    scale2 = jnp.concatenate([scale, scale], axis=1)
    shift2 = jnp.concatenate([shift, shift], axis=1)
    o_ref[...] = x_ref[...] + h_ref[...] * scale2 + shift2

  return pl.pallas_call(
      body,
      grid=(GRID,),
      in_specs=[pl.BlockSpec((BLK2, 128), lambda i: (i, 0)),
                pl.BlockSpec((BLK2, 128), lambda i: (i, 0)),
                pl.BlockSpec((1, 128), lambda i: (0, 0)),
                pl.BlockSpec((1, 128), lambda i: (0, 0)),
                pl.BlockSpec((1, D), lambda i: (0, 0)),
                pl.BlockSpec((1, D), lambda i: (0, 0))],
      out_specs=pl.BlockSpec((BLK2, 128), lambda i: (i, 0)),
      out_shape=jax.ShapeDtypeStruct((NP2, 128), jnp.float32),
  )(h2, x2, ssum, sqsum, gamma, beta)


def _final(hg1p, hg2p, cnt4, w, b):
  def body(*refs):
    parts = refs[:16]
    cnt0, cnt1, w_ref, b_ref, o_ref = refs[16:]
    cnt = cnt0[0, :, 0:1] + cnt1[0, :, 0:1]
    inv = 1.0 / jnp.maximum(cnt, 1.0)
    o = b_ref[...] * jnp.ones((G, 1), jnp.float32)
    for q in range(4):
      sq = (parts[q][0] + parts[4 + q][0] + parts[8 + q][0]
            + parts[12 + q][0]) * inv
      o = o + jnp.dot(sq, w_ref[q * Q:(q + 1) * Q, :],
                      preferred_element_type=jnp.float32)
    o_ref[...] = o

  qspecs = [pl.BlockSpec((1, G, Q), lambda i, c=c, q=q: (c, q, 0))
            for c in range(2) for q in range(4)]
  return pl.pallas_call(
      body,
      grid=(1,),
      in_specs=qspecs + qspecs +
      [pl.BlockSpec((1, G, 8), lambda i: (0, 0, 0)),
       pl.BlockSpec((1, G, 8), lambda i: (1, 0, 0)),
       pl.BlockSpec((D, D), lambda i: (0, 0)),
       pl.BlockSpec((1, D), lambda i: (0, 0))],
      out_specs=pl.BlockSpec((G, D), lambda i: (0, 0)),
      out_shape=jax.ShapeDtypeStruct((G, D), jnp.float32),
  )(*([hg1p] * 8), *([hg2p] * 8), cnt4, cnt4, w, b)


def _pack_w(w, b):
  w2 = jnp.zeros((128, 128), jnp.float32)
  w2 = w2.at[:D, :D].set(w).at[D:, D:].set(w)
  b2 = jnp.concatenate([b, b]).reshape(1, 128)
  return w2, b2


def kernel(nodes_feat, edge_index, edges_feat, nodes_num_norm_sqrt,
           edges_num_norm_sqrt, graph_ids, W_emb, b_emb, W1, b1, gamma1,
           beta1, Wo, bo, gamma_o, beta_o, W_out, b_out):
  src = edge_index[0]
  dst = edge_index[1]
  pad = EP - E
  srcP = 4 * jnp.concatenate([src, jnp.zeros((pad,), jnp.int32)])
  dstP = jnp.concatenate([dst, jnp.full((pad,), N, jnp.int32)])
  src4 = jnp.stack([srcP, srcP + 1, srcP + 2,
                    srcP + 3]).reshape(2, 2, ROWS, 128)
  dst2 = dstP.reshape(ROWS, 128)
  gidQ = 256 * jnp.tile(jnp.arange(4, dtype=jnp.int32), NP)
  gidN = jnp.repeat(
      jnp.concatenate([graph_ids, jnp.full((NP - N,), -1, jnp.int32)]), 4)
  gidx = jnp.where(gidN >= 0, gidQ + gidN,
                   1024 + jnp.tile(jnp.arange(4, dtype=jnp.int32), NP))
  gidx2 = gidx.reshape(PROWS, 128)
  z16 = jnp.zeros((ZCH, Q), jnp.float32)
  z8 = jnp.zeros((ZCH, 8), jnp.float32)
  ones16 = jnp.ones((128, Q), jnp.float32)
  ones8 = jnp.ones((128, 8), jnp.float32)

  nf2 = nodes_feat.reshape(N // 2, 2 * D)
  snP = jnp.concatenate([nodes_num_norm_sqrt,
                         jnp.zeros((NP - N, 1), jnp.float32)])
  sn2 = jnp.broadcast_to(snP, (NP, D)).reshape(NP2, 128)
  w_emb2, b_emb2 = _pack_w(W_emb, b_emb)
  w12, b12 = _pack_w(W1, b1)
  wo2, bo2 = _pack_w(Wo, bo)

  e0 = _emb_call(nf2, w_emb2, b_emb2)
  msg1, deg2 = _edge_deg(src4, dst2, e0.reshape(4 * NP, Q), z16, ones16)
  degp = deg2.reshape(2, NP2, 128)
  h1, s1, ss1 = _layer_a(msg1.reshape(NP2, 128), degp, degp, sn2, w12, b12)
  e1 = _layer_b(h1, e0, s1, ss1, gamma1.reshape(1, D), beta1.reshape(1, D))
  e1v = e1.reshape(4 * NP, Q)
  msg2, hg1p, cnt4 = _edge_pool(src4, dst2, e1v, z16, z8, gidx2, ones8)
  h2, s2, ss2 = _layer_a(msg2.reshape(NP2, 128), degp, degp, sn2, wo2, bo2)
  e2 = _layer_b(h2, e1, s2, ss2, gamma_o.reshape(1, D),
                beta_o.reshape(1, D))
  hg2p = _pool2(e2.reshape(4 * NP, Q), gidx2, z16)
  return _final(hg1p, hg2p, cnt4, W_out, b_out.reshape(1, D))
